# Initial kernel scaffold; baseline (speedup 1.0000x reference)
import jax
import jax.numpy as jnp
from jax.experimental import pallas as pl


def kernel(x_c, L, conv_B, conv_C, alpha):
    raise NotImplementedError("write your pallas kernel here")


# speedup vs baseline: 1.2259x; 1.2259x over previous
import jax
import jax.numpy as jnp
from jax.experimental import pallas as pl

_B, _N, _E, _D = 2, 2048, 16384, 16
_BD = _B * _D
_BE = 2048
_JE = _E // _BE
_ITERS = 4


def _lpass_kernel(alpha_ref, L_ref, y_ref, w_ref):
    a = alpha_ref[0, 0]
    Lb = L_ref[...]
    s = jax.lax.dot_general(Lb, y_ref[...], (((0,), (0,)), ((), ())),
                            preferred_element_type=jnp.float32)
    z = jnp.where(s > a, s - a, jnp.where(s < -a, s + a, jnp.zeros_like(s)))
    u = jnp.dot(Lb, z, preferred_element_type=jnp.float32)

    @pl.when(pl.program_id(0) == 0)
    def _init():
        w_ref[...] = u

    @pl.when(pl.program_id(0) != 0)
    def _acc():
        w_ref[...] += u


def _mm_kernel(C_ref, v_ref, o_ref):
    o_ref[0] = jnp.dot(C_ref[0], v_ref[0], preferred_element_type=jnp.float32)


def _mm_add_kernel(C_ref, v_ref, a_ref, o_ref):
    o_ref[0] = (jnp.dot(C_ref[0], v_ref[0], preferred_element_type=jnp.float32)
                + a_ref[0])


def _conv_mm(C, v, add=None):
    spec_c = pl.BlockSpec((1, _N, _N), lambda b: (b, 0, 0))
    spec_v = pl.BlockSpec((1, _N, _D), lambda b: (b, 0, 0))
    out_shape = jax.ShapeDtypeStruct((_B, _N, _D), jnp.float32)
    if add is None:
        return pl.pallas_call(
            _mm_kernel, grid=(_B,),
            in_specs=[spec_c, spec_v], out_specs=spec_v, out_shape=out_shape,
        )(C, v)
    return pl.pallas_call(
        _mm_add_kernel, grid=(_B,),
        in_specs=[spec_c, spec_v, spec_v], out_specs=spec_v, out_shape=out_shape,
    )(C, v, add)


def _lpass(alpha2, L, y2):
    return pl.pallas_call(
        _lpass_kernel,
        grid=(_JE,),
        in_specs=[
            pl.BlockSpec((1, 1), lambda j: (0, 0)),
            pl.BlockSpec((_N, _BE), lambda j: (0, j)),
            pl.BlockSpec((_N, _BD), lambda j: (0, 0)),
        ],
        out_specs=pl.BlockSpec((_N, _BD), lambda j: (0, 0)),
        out_shape=jax.ShapeDtypeStruct((_N, _BD), jnp.float32),
    )(alpha2, L, y2)


def kernel(x_c, L, conv_B, conv_C, alpha):
    alpha2 = alpha.reshape(1, 1)

    bx = _conv_mm(conv_B, x_c)
    y = bx
    for _ in range(_ITERS - 1):
        y2 = y.transpose(1, 0, 2).reshape(_N, _BD)
        w2 = _lpass(alpha2, L, y2)
        w = w2.reshape(_N, _B, _D).transpose(1, 0, 2)
        y = _conv_mm(conv_C, w, bx)
    return y


# speedup vs baseline: 1.2278x; 1.0016x over previous
import jax
import jax.numpy as jnp
from jax.experimental import pallas as pl

_B, _N, _E, _D = 2, 2048, 16384, 16
_BD = _B * _D
_BE = 2048
_JE = _E // _BE
_ITERS = 4


def _lpass_kernel(alpha_ref, L_ref, y_ref, w_ref):
    a = alpha_ref[0, 0]
    Lb = L_ref[...].astype(jnp.bfloat16)
    yb = y_ref[...].astype(jnp.bfloat16)
    s = jax.lax.dot_general(Lb, yb, (((0,), (0,)), ((), ())),
                            preferred_element_type=jnp.float32)
    z = jnp.where(s > a, s - a, jnp.where(s < -a, s + a, jnp.zeros_like(s)))
    u = jnp.dot(Lb, z.astype(jnp.bfloat16), preferred_element_type=jnp.float32)

    @pl.when(pl.program_id(0) == 0)
    def _init():
        w_ref[...] = u

    @pl.when(pl.program_id(0) != 0)
    def _acc():
        w_ref[...] += u


def _mm_kernel(C_ref, v_ref, o_ref):
    o_ref[0] = jnp.dot(C_ref[0], v_ref[0], preferred_element_type=jnp.float32)


def _mm_add_kernel(C_ref, v_ref, a_ref, o_ref):
    o_ref[0] = (jnp.dot(C_ref[0], v_ref[0], preferred_element_type=jnp.float32)
                + a_ref[0])


def _conv_mm(C, v, add=None):
    spec_c = pl.BlockSpec((1, _N, _N), lambda b: (b, 0, 0))
    spec_v = pl.BlockSpec((1, _N, _D), lambda b: (b, 0, 0))
    out_shape = jax.ShapeDtypeStruct((_B, _N, _D), jnp.float32)
    if add is None:
        return pl.pallas_call(
            _mm_kernel, grid=(_B,),
            in_specs=[spec_c, spec_v], out_specs=spec_v, out_shape=out_shape,
        )(C, v)
    return pl.pallas_call(
        _mm_add_kernel, grid=(_B,),
        in_specs=[spec_c, spec_v, spec_v], out_specs=spec_v, out_shape=out_shape,
    )(C, v, add)


def _lpass(alpha2, L, y2):
    return pl.pallas_call(
        _lpass_kernel,
        grid=(_JE,),
        in_specs=[
            pl.BlockSpec((1, 1), lambda j: (0, 0)),
            pl.BlockSpec((_N, _BE), lambda j: (0, j)),
            pl.BlockSpec((_N, _BD), lambda j: (0, 0)),
        ],
        out_specs=pl.BlockSpec((_N, _BD), lambda j: (0, 0)),
        out_shape=jax.ShapeDtypeStruct((_N, _BD), jnp.float32),
    )(alpha2, L, y2)


def kernel(x_c, L, conv_B, conv_C, alpha):
    alpha2 = alpha.reshape(1, 1)

    bx = _conv_mm(conv_B, x_c)
    y = bx
    for _ in range(_ITERS - 1):
        y2 = y.transpose(1, 0, 2).reshape(_N, _BD)
        w2 = _lpass(alpha2, L, y2)
        w = w2.reshape(_N, _B, _D).transpose(1, 0, 2)
        y = _conv_mm(conv_C, w, bx)
    return y


# speedup vs baseline: 1.3863x; 1.1291x over previous
import jax
import jax.numpy as jnp
from jax.experimental import pallas as pl
from jax.experimental.pallas import tpu as pltpu

_B, _N, _E, _D = 2, 2048, 16384, 16
_BD = _B * _D
_BE = 1024
_JE = _E // _BE
_ITERS = 4


def _soft(s, a):
    return jnp.where(s > a, s - a, jnp.where(s < -a, s + a, jnp.zeros_like(s)))


def _main_kernel(alpha_ref, L_ref, C_ref, bx_ref, out_ref, y_scr, w_scr):
    t = pl.program_id(0)
    j = pl.program_id(1)

    @pl.when((t == 0) & (j == 0))
    def _seed():
        y_scr[...] = bx_ref[...]

    @pl.when(j < _JE)
    def _lpass():
        a = alpha_ref[0, 0]
        Lb = L_ref[...].astype(jnp.bfloat16)
        yb = y_scr[...].astype(jnp.bfloat16)
        s = jax.lax.dot_general(Lb, yb, (((0,), (0,)), ((), ())),
                                preferred_element_type=jnp.float32)
        u = jnp.dot(Lb, _soft(s, a).astype(jnp.bfloat16),
                    preferred_element_type=jnp.float32)

        @pl.when(j == 0)
        def _init():
            w_scr[...] = u

        @pl.when(j != 0)
        def _acc():
            w_scr[...] += u

    @pl.when(j == _JE)
    def _conv():
        wb = w_scr[...].astype(jnp.bfloat16)
        parts = []
        for b in range(_B):
            parts.append(jnp.dot(C_ref[b], wb[:, b * _D:(b + 1) * _D],
                                 preferred_element_type=jnp.float32))
        y_new = jnp.concatenate(parts, axis=1) + bx_ref[...]
        y_scr[...] = y_new

        @pl.when(t == _ITERS - 2)
        def _emit():
            out_ref[...] = y_new


def _bx_kernel(Cb_ref, x_ref, o_ref):
    parts = []
    for b in range(_B):
        parts.append(jnp.dot(Cb_ref[b], x_ref[:, b * _D:(b + 1) * _D],
                             preferred_element_type=jnp.float32))
    o_ref[...] = jnp.concatenate(parts, axis=1)


def kernel(x_c, L, conv_B, conv_C, alpha):
    alpha2 = alpha.reshape(1, 1)
    x2 = x_c.transpose(1, 0, 2).reshape(_N, _BD)
    C16 = conv_C.astype(jnp.bfloat16)

    bx2 = pl.pallas_call(
        _bx_kernel,
        in_specs=[pl.BlockSpec((_B, _N, _N), lambda: (0, 0, 0)),
                  pl.BlockSpec((_N, _BD), lambda: (0, 0))],
        out_specs=pl.BlockSpec((_N, _BD), lambda: (0, 0)),
        out_shape=jax.ShapeDtypeStruct((_N, _BD), jnp.float32),
    )(conv_B, x2)

    y2 = pl.pallas_call(
        _main_kernel,
        grid=(_ITERS - 1, _JE + 1),
        in_specs=[
            pl.BlockSpec((1, 1), lambda t, j: (0, 0)),
            pl.BlockSpec((_N, _BE), lambda t, j: (0, jnp.minimum(j, _JE - 1))),
            pl.BlockSpec((_B, _N, _N), lambda t, j: (0, 0, 0)),
            pl.BlockSpec((_N, _BD), lambda t, j: (0, 0)),
        ],
        out_specs=pl.BlockSpec((_N, _BD), lambda t, j: (0, 0)),
        out_shape=jax.ShapeDtypeStruct((_N, _BD), jnp.float32),
        scratch_shapes=[pltpu.VMEM((_N, _BD), jnp.float32),
                        pltpu.VMEM((_N, _BD), jnp.float32)],
    )(alpha2, L, C16, bx2)

    return y2.reshape(_N, _B, _D).transpose(1, 0, 2)


# speedup vs baseline: 2.0468x; 1.4765x over previous
import jax
import jax.numpy as jnp
from jax.experimental import pallas as pl
from jax.experimental.pallas import tpu as pltpu

_B, _N, _E, _D = 2, 2048, 16384, 16
_BD = _B * _D
_BE = 1024
_JE = _E // _BE
_ITERS = 4


def _soft(s, a):
    return jnp.where(s > a, s - a, jnp.where(s < -a, s + a, jnp.zeros_like(s)))


def _main_kernel(alpha_ref, L_ref, C_ref, bx_ref, out_ref, y_scr, w_scr):
    t = pl.program_id(0)
    j = pl.program_id(1)

    @pl.when((t == 0) & (j == 0))
    def _seed():
        y_scr[...] = bx_ref[...]

    @pl.when(j < _JE)
    def _lpass():
        a = alpha_ref[0, 0]
        Lb = L_ref[...].astype(jnp.bfloat16)
        yb = y_scr[...].astype(jnp.bfloat16)
        s = jax.lax.dot_general(yb, Lb, (((1,), (0,)), ((), ())),
                                preferred_element_type=jnp.float32)
        u = jax.lax.dot_general(_soft(s, a).astype(jnp.bfloat16), Lb,
                                (((1,), (1,)), ((), ())),
                                preferred_element_type=jnp.float32)

        @pl.when(j == 0)
        def _init():
            w_scr[...] = u

        @pl.when(j != 0)
        def _acc():
            w_scr[...] += u

    @pl.when(j == _JE)
    def _conv():
        wb = w_scr[...].astype(jnp.bfloat16)
        parts = []
        for b in range(_B):
            parts.append(jax.lax.dot_general(
                wb[b * _D:(b + 1) * _D, :], C_ref[b],
                (((1,), (1,)), ((), ())),
                preferred_element_type=jnp.float32))
        y_new = jnp.concatenate(parts, axis=0) + bx_ref[...]
        y_scr[...] = y_new

        @pl.when(t == _ITERS - 2)
        def _emit():
            out_ref[...] = y_new


def _bx_kernel(Cb_ref, x_ref, o_ref):
    parts = []
    for b in range(_B):
        parts.append(jax.lax.dot_general(
            x_ref[b * _D:(b + 1) * _D, :], Cb_ref[b],
            (((1,), (1,)), ((), ())),
            preferred_element_type=jnp.float32))
    o_ref[...] = jnp.concatenate(parts, axis=0)


def kernel(x_c, L, conv_B, conv_C, alpha):
    alpha2 = alpha.reshape(1, 1)
    x2 = x_c.transpose(0, 2, 1).reshape(_BD, _N)
    C16 = conv_C.astype(jnp.bfloat16)

    bx2 = pl.pallas_call(
        _bx_kernel,
        in_specs=[pl.BlockSpec((_B, _N, _N), lambda: (0, 0, 0)),
                  pl.BlockSpec((_BD, _N), lambda: (0, 0))],
        out_specs=pl.BlockSpec((_BD, _N), lambda: (0, 0)),
        out_shape=jax.ShapeDtypeStruct((_BD, _N), jnp.float32),
    )(conv_B, x2)

    y2 = pl.pallas_call(
        _main_kernel,
        grid=(_ITERS - 1, _JE + 1),
        in_specs=[
            pl.BlockSpec((1, 1), lambda t, j: (0, 0)),
            pl.BlockSpec((_N, _BE), lambda t, j: (0, jnp.minimum(j, _JE - 1))),
            pl.BlockSpec((_B, _N, _N), lambda t, j: (0, 0, 0)),
            pl.BlockSpec((_BD, _N), lambda t, j: (0, 0)),
        ],
        out_specs=pl.BlockSpec((_BD, _N), lambda t, j: (0, 0)),
        out_shape=jax.ShapeDtypeStruct((_BD, _N), jnp.float32),
        scratch_shapes=[pltpu.VMEM((_BD, _N), jnp.float32),
                        pltpu.VMEM((_BD, _N), jnp.float32)],
    )(alpha2, L, C16, bx2)

    return y2.reshape(_B, _D, _N).transpose(0, 2, 1)
